# baseline (speedup 1.0000x reference)
import functools

import jax
import jax.numpy as jnp
from jax import lax
from jax.experimental import pallas as pl
from jax.experimental.pallas import tpu as pltpu
from jax.experimental.pallas import tpu_sc as plsc

E = 16
D = 1024
H = 4096
O = 1024
N = 8192
T = 640
NTILES = (N + T - 1) // T + E - 1
P = NTILES * T

NW = 32


def _sc_scatter_rows(table, idx3):
    n_words = table.shape[1]
    k, chunk = idx3.shape[1], idx3.shape[2]
    per_w = k * chunk
    mesh = plsc.VectorSubcoreMesh(core_axis_name="c", subcore_axis_name="s")

    def body(table_hbm, idx_hbm, out_hbm, idx_v, rows_v, sem):
        wid = lax.axis_index("s") * 2 + lax.axis_index("c")
        base = wid * per_w
        pltpu.sync_copy(idx_hbm.at[wid], idx_v)

        def one_chunk(j, carry):
            pltpu.sync_copy(table_hbm.at[pl.ds(base + j * chunk, chunk)], rows_v)
            pltpu.async_copy(rows_v, out_hbm.at[idx_v.at[j]], sem).wait()
            return carry

        lax.fori_loop(0, k, one_chunk, 0)

    return pl.kernel(
        body,
        mesh=mesh,
        out_type=jax.ShapeDtypeStruct((P, n_words), jnp.float32),
        scratch_types=[
            pltpu.VMEM((k, chunk), jnp.int32),
            pltpu.VMEM((chunk, n_words), jnp.float32),
            pltpu.SemaphoreType.DMA,
        ],
    )(table, idx3)


def _sc_gather_rows(table, idx, chunk):
    n_rows = idx.shape[0]
    n_words = table.shape[1]
    per_w = n_rows // NW
    n_chunks = per_w // chunk
    mesh = plsc.VectorSubcoreMesh(core_axis_name="c", subcore_axis_name="s")

    def body(table_hbm, idx_hbm, out_hbm, idx_v, rows_v, sem):
        wid = lax.axis_index("s") * 2 + lax.axis_index("c")
        base = wid * per_w

        def one_chunk(i, carry):
            off = base + i * chunk
            pltpu.sync_copy(idx_hbm.at[pl.ds(off, chunk)], idx_v)
            pltpu.async_copy(table_hbm.at[idx_v], rows_v, sem).wait()
            pltpu.sync_copy(rows_v, out_hbm.at[pl.ds(off, chunk)])
            return carry

        lax.fori_loop(0, n_chunks, one_chunk, 0)

    return pl.kernel(
        body,
        mesh=mesh,
        out_type=jax.ShapeDtypeStruct((n_rows, n_words), jnp.float32),
        scratch_types=[
            pltpu.VMEM((chunk,), jnp.int32),
            pltpu.VMEM((chunk, n_words), jnp.float32),
            pltpu.SemaphoreType.DMA,
        ],
    )(table, idx)


NC = 8
HCC = H // NC


def _fused_body(em_ref, nt_ref, x_ref, w1_ref, w2_ref, out_ref):
    t = pl.program_id(0)
    c = pl.program_id(1)

    @pl.when(t < nt_ref[0])
    def _():
        h = lax.dot_general(
            x_ref[...], w1_ref[0],
            (((1,), (1,)), ((), ())),
            preferred_element_type=jnp.float32,
        )
        part = lax.dot_general(
            h, w2_ref[0],
            (((1,), (1,)), ((), ())),
            preferred_element_type=jnp.float32,
        )

        @pl.when(c == 0)
        def _():
            out_ref[...] = part

        @pl.when(c > 0)
        def _():
            out_ref[...] += part


def _c_eff(t, c, nt):
    zig = jnp.where(t % 2 == 0, c, NC - 1 - c)
    frozen = jnp.where((nt - 1) % 2 == 0, NC - 1, 0)
    return jnp.where(t < nt, zig, frozen)


def _moe_fused(x_pad, w1, w2, expert_map, n_tiles):
    grid_spec = pltpu.PrefetchScalarGridSpec(
        num_scalar_prefetch=2,
        grid=(NTILES, NC),
        in_specs=[
            pl.BlockSpec(
                (T, D), lambda t, c, em, nt: (jnp.minimum(t, nt[0] - 1), 0)),
            pl.BlockSpec(
                (1, HCC, D),
                lambda t, c, em, nt: (em[t], _c_eff(t, c, nt[0]), 0)),
            pl.BlockSpec(
                (1, O, HCC),
                lambda t, c, em, nt: (em[t], 0, _c_eff(t, c, nt[0]))),
        ],
        out_specs=pl.BlockSpec((T, O), lambda t, c, em, nt: (t, 0)),
    )
    return pl.pallas_call(
        _fused_body,
        grid_spec=grid_spec,
        out_shape=jax.ShapeDtypeStruct((P, O), jnp.float32),
        compiler_params=pltpu.CompilerParams(
            vmem_limit_bytes=56 * 1024 * 1024),
    )(expert_map, n_tiles, x_pad, w1, w2)


def kernel(inp, gate, weight1, weight2):
    gate = gate.astype(jnp.int32)

    onehot = (gate[:, None] == jnp.arange(E, dtype=jnp.int32)[None, :])
    onehot_i = onehot.astype(jnp.int32)
    incl = jnp.cumsum(onehot_i, axis=0)
    counts = incl[-1]
    rank = jnp.sum(jnp.where(onehot, incl, 0), axis=1) - 1
    tiles_per_e = (counts + T - 1) // T
    tile_start = jnp.concatenate([jnp.zeros((1,), jnp.int32),
                                  jnp.cumsum(tiles_per_e)[:-1]]).astype(jnp.int32)
    n_tiles = tile_start[-1] + tiles_per_e[-1]
    pad_off = tile_start * T
    pad_pos = jnp.sum(jnp.where(onehot, pad_off[None, :], 0), axis=1) + rank
    pad_pos = pad_pos.astype(jnp.int32)
    t_ids = jnp.arange(NTILES, dtype=jnp.int32)
    expert_map = jnp.clip(
        jnp.sum((t_ids[:, None] >= tile_start[None, :]).astype(jnp.int32),
                axis=1) - 1, 0, E - 1).astype(jnp.int32)
    expert_map = expert_map[jnp.minimum(t_ids, n_tiles - 1)]

    idx3 = pad_pos.reshape(NW, 4, N // NW // 4)
    x_pad = _sc_scatter_rows(inp, idx3)

    nt = n_tiles.reshape(1)
    out_pad = _moe_fused(x_pad, weight1, weight2, expert_map, nt)

    return _sc_gather_rows(out_pad, pad_pos, 64)


# speedup vs baseline: 1.2330x; 1.2330x over previous
import functools

import jax
import jax.numpy as jnp
from jax import lax
from jax.experimental import pallas as pl
from jax.experimental.pallas import tpu as pltpu
from jax.experimental.pallas import tpu_sc as plsc

E = 16
D = 1024
H = 4096
O = 1024
N = 8192
T = 640
NTILES = (N + T - 1) // T + E - 1
P = NTILES * T

NW = 32


def _sc_scatter_rows(table, idx3):
    n_words = table.shape[1]
    k, chunk = idx3.shape[1], idx3.shape[2]
    per_w = k * chunk
    mesh = plsc.VectorSubcoreMesh(core_axis_name="c", subcore_axis_name="s")

    def body(table_hbm, idx_hbm, out_hbm, idx_v, rows_v, sem):
        wid = lax.axis_index("s") * 2 + lax.axis_index("c")
        base = wid * per_w
        pltpu.sync_copy(idx_hbm.at[wid], idx_v)

        def one_chunk(j, carry):
            pltpu.sync_copy(table_hbm.at[pl.ds(base + j * chunk, chunk)], rows_v)
            pltpu.async_copy(rows_v, out_hbm.at[idx_v.at[j]], sem).wait()
            return carry

        lax.fori_loop(0, k, one_chunk, 0)

    return pl.kernel(
        body,
        mesh=mesh,
        out_type=jax.ShapeDtypeStruct((P, n_words), jnp.float32),
        scratch_types=[
            pltpu.VMEM((k, chunk), jnp.int32),
            pltpu.VMEM((chunk, n_words), jnp.float32),
            pltpu.SemaphoreType.DMA,
        ],
    )(table, idx3)


def _sc_gather_rows(table, idx, chunk):
    n_rows = idx.shape[0]
    n_words = table.shape[1]
    per_w = n_rows // NW
    n_chunks = per_w // chunk
    mesh = plsc.VectorSubcoreMesh(core_axis_name="c", subcore_axis_name="s")

    def body(table_hbm, idx_hbm, out_hbm, idx_v, rows_v, sem):
        wid = lax.axis_index("s") * 2 + lax.axis_index("c")
        base = wid * per_w

        def one_chunk(i, carry):
            off = base + i * chunk
            pltpu.sync_copy(idx_hbm.at[pl.ds(off, chunk)], idx_v)
            pltpu.async_copy(table_hbm.at[idx_v], rows_v, sem).wait()
            pltpu.sync_copy(rows_v, out_hbm.at[pl.ds(off, chunk)])
            return carry

        lax.fori_loop(0, n_chunks, one_chunk, 0)

    return pl.kernel(
        body,
        mesh=mesh,
        out_type=jax.ShapeDtypeStruct((n_rows, n_words), jnp.float32),
        scratch_types=[
            pltpu.VMEM((chunk,), jnp.int32),
            pltpu.VMEM((chunk, n_words), jnp.float32),
            pltpu.SemaphoreType.DMA,
        ],
    )(table, idx)


NC = 2
HCC = H // NC


def _fused_body(em_ref, nt_ref, x_ref, w1_ref, w2_ref, out_ref):
    t = pl.program_id(0)
    c = pl.program_id(1)

    @pl.when(t < nt_ref[0])
    def _():
        h = lax.dot_general(
            x_ref[...], w1_ref[0],
            (((1,), (1,)), ((), ())),
            preferred_element_type=jnp.float32,
        )
        part = lax.dot_general(
            h, w2_ref[0],
            (((1,), (1,)), ((), ())),
            preferred_element_type=jnp.float32,
        )

        @pl.when(c == 0)
        def _():
            out_ref[...] = part

        @pl.when(c > 0)
        def _():
            out_ref[...] += part


def _c_eff(t, c, nt):
    zig = jnp.where(t % 2 == 0, c, NC - 1 - c)
    frozen = jnp.where((nt - 1) % 2 == 0, NC - 1, 0)
    return jnp.where(t < nt, zig, frozen)


def _moe_fused(x_pad, w1, w2, expert_map, n_tiles):
    grid_spec = pltpu.PrefetchScalarGridSpec(
        num_scalar_prefetch=2,
        grid=(NTILES, NC),
        in_specs=[
            pl.BlockSpec(
                (T, D), lambda t, c, em, nt: (jnp.minimum(t, nt[0] - 1), 0)),
            pl.BlockSpec(
                (1, HCC, D),
                lambda t, c, em, nt: (em[t], _c_eff(t, c, nt[0]), 0)),
            pl.BlockSpec(
                (1, O, HCC),
                lambda t, c, em, nt: (em[t], 0, _c_eff(t, c, nt[0]))),
        ],
        out_specs=pl.BlockSpec((T, O), lambda t, c, em, nt: (t, 0)),
    )
    return pl.pallas_call(
        _fused_body,
        grid_spec=grid_spec,
        out_shape=jax.ShapeDtypeStruct((P, O), jnp.float32),
        compiler_params=pltpu.CompilerParams(
            vmem_limit_bytes=56 * 1024 * 1024),
    )(expert_map, n_tiles, x_pad, w1, w2)


def kernel(inp, gate, weight1, weight2):
    gate = gate.astype(jnp.int32)

    onehot = (gate[:, None] == jnp.arange(E, dtype=jnp.int32)[None, :])
    onehot_i = onehot.astype(jnp.int32)
    incl = jnp.cumsum(onehot_i, axis=0)
    counts = incl[-1]
    rank = jnp.sum(jnp.where(onehot, incl, 0), axis=1) - 1
    tiles_per_e = (counts + T - 1) // T
    tile_start = jnp.concatenate([jnp.zeros((1,), jnp.int32),
                                  jnp.cumsum(tiles_per_e)[:-1]]).astype(jnp.int32)
    n_tiles = tile_start[-1] + tiles_per_e[-1]
    pad_off = tile_start * T
    pad_pos = jnp.sum(jnp.where(onehot, pad_off[None, :], 0), axis=1) + rank
    pad_pos = pad_pos.astype(jnp.int32)
    t_ids = jnp.arange(NTILES, dtype=jnp.int32)
    expert_map = jnp.clip(
        jnp.sum((t_ids[:, None] >= tile_start[None, :]).astype(jnp.int32),
                axis=1) - 1, 0, E - 1).astype(jnp.int32)
    expert_map = expert_map[jnp.minimum(t_ids, n_tiles - 1)]

    idx3 = pad_pos.reshape(NW, 4, N // NW // 4)
    x_pad = _sc_scatter_rows(inp, idx3)

    nt = n_tiles.reshape(1)
    out_pad = _moe_fused(x_pad, weight1, weight2, expert_map, nt)

    return _sc_gather_rows(out_pad, pad_pos, 64)


# speedup vs baseline: 1.2733x; 1.0326x over previous
import functools

import jax
import jax.numpy as jnp
from jax import lax
from jax.experimental import pallas as pl
from jax.experimental.pallas import tpu as pltpu
from jax.experimental.pallas import tpu_sc as plsc

E = 16
D = 1024
H = 4096
O = 1024
N = 8192
T = 576
NTILES = (N + T - 1) // T + E - 1
P = NTILES * T

NW = 32


def _sc_scatter_rows(table, idx3):
    n_words = table.shape[1]
    k, chunk = idx3.shape[1], idx3.shape[2]
    per_w = k * chunk
    mesh = plsc.VectorSubcoreMesh(core_axis_name="c", subcore_axis_name="s")

    def body(table_hbm, idx_hbm, out_hbm, idx_v, rows_v, sem):
        wid = lax.axis_index("s") * 2 + lax.axis_index("c")
        base = wid * per_w
        pltpu.sync_copy(idx_hbm.at[wid], idx_v)

        def one_chunk(j, carry):
            pltpu.sync_copy(table_hbm.at[pl.ds(base + j * chunk, chunk)], rows_v)
            pltpu.async_copy(rows_v, out_hbm.at[idx_v.at[j]], sem).wait()
            return carry

        lax.fori_loop(0, k, one_chunk, 0)

    return pl.kernel(
        body,
        mesh=mesh,
        out_type=jax.ShapeDtypeStruct((P, n_words), jnp.float32),
        scratch_types=[
            pltpu.VMEM((k, chunk), jnp.int32),
            pltpu.VMEM((chunk, n_words), jnp.float32),
            pltpu.SemaphoreType.DMA,
        ],
    )(table, idx3)


def _sc_gather_rows(table, idx, chunk):
    n_rows = idx.shape[0]
    n_words = table.shape[1]
    per_w = n_rows // NW
    n_chunks = per_w // chunk
    mesh = plsc.VectorSubcoreMesh(core_axis_name="c", subcore_axis_name="s")

    def body(table_hbm, idx_hbm, out_hbm, idx_v, rows_v, sem):
        wid = lax.axis_index("s") * 2 + lax.axis_index("c")
        base = wid * per_w

        def one_chunk(i, carry):
            off = base + i * chunk
            pltpu.sync_copy(idx_hbm.at[pl.ds(off, chunk)], idx_v)
            pltpu.async_copy(table_hbm.at[idx_v], rows_v, sem).wait()
            pltpu.sync_copy(rows_v, out_hbm.at[pl.ds(off, chunk)])
            return carry

        lax.fori_loop(0, n_chunks, one_chunk, 0)

    return pl.kernel(
        body,
        mesh=mesh,
        out_type=jax.ShapeDtypeStruct((n_rows, n_words), jnp.float32),
        scratch_types=[
            pltpu.VMEM((chunk,), jnp.int32),
            pltpu.VMEM((chunk, n_words), jnp.float32),
            pltpu.SemaphoreType.DMA,
        ],
    )(table, idx)


NC = 2
HCC = H // NC


def _fused_body(em_ref, nt_ref, x_ref, w1_ref, w2_ref, out_ref):
    t = pl.program_id(0)
    c = pl.program_id(1)

    @pl.when(t < nt_ref[0])
    def _():
        h = lax.dot_general(
            x_ref[...], w1_ref[0],
            (((1,), (1,)), ((), ())),
            preferred_element_type=jnp.float32,
        )
        part = lax.dot_general(
            h, w2_ref[0],
            (((1,), (1,)), ((), ())),
            preferred_element_type=jnp.float32,
        )

        @pl.when(c == 0)
        def _():
            out_ref[...] = part

        @pl.when(c > 0)
        def _():
            out_ref[...] += part


def _c_eff(t, c, nt):
    zig = jnp.where(t % 2 == 0, c, NC - 1 - c)
    frozen = jnp.where((nt - 1) % 2 == 0, NC - 1, 0)
    return jnp.where(t < nt, zig, frozen)


def _moe_fused(x_pad, w1, w2, expert_map, n_tiles):
    grid_spec = pltpu.PrefetchScalarGridSpec(
        num_scalar_prefetch=2,
        grid=(NTILES, NC),
        in_specs=[
            pl.BlockSpec(
                (T, D), lambda t, c, em, nt: (jnp.minimum(t, nt[0] - 1), 0)),
            pl.BlockSpec(
                (1, HCC, D),
                lambda t, c, em, nt: (em[t], _c_eff(t, c, nt[0]), 0)),
            pl.BlockSpec(
                (1, O, HCC),
                lambda t, c, em, nt: (em[t], 0, _c_eff(t, c, nt[0]))),
        ],
        out_specs=pl.BlockSpec((T, O), lambda t, c, em, nt: (t, 0)),
    )
    return pl.pallas_call(
        _fused_body,
        grid_spec=grid_spec,
        out_shape=jax.ShapeDtypeStruct((P, O), jnp.float32),
        compiler_params=pltpu.CompilerParams(
            vmem_limit_bytes=56 * 1024 * 1024),
    )(expert_map, n_tiles, x_pad, w1, w2)


def kernel(inp, gate, weight1, weight2):
    gate = gate.astype(jnp.int32)

    onehot = (gate[:, None] == jnp.arange(E, dtype=jnp.int32)[None, :])
    onehot_i = onehot.astype(jnp.int32)
    incl = jnp.cumsum(onehot_i, axis=0)
    counts = incl[-1]
    rank = jnp.sum(jnp.where(onehot, incl, 0), axis=1) - 1
    tiles_per_e = (counts + T - 1) // T
    tile_start = jnp.concatenate([jnp.zeros((1,), jnp.int32),
                                  jnp.cumsum(tiles_per_e)[:-1]]).astype(jnp.int32)
    n_tiles = tile_start[-1] + tiles_per_e[-1]
    pad_off = tile_start * T
    pad_pos = jnp.sum(jnp.where(onehot, pad_off[None, :], 0), axis=1) + rank
    pad_pos = pad_pos.astype(jnp.int32)
    t_ids = jnp.arange(NTILES, dtype=jnp.int32)
    expert_map = jnp.clip(
        jnp.sum((t_ids[:, None] >= tile_start[None, :]).astype(jnp.int32),
                axis=1) - 1, 0, E - 1).astype(jnp.int32)
    expert_map = expert_map[jnp.minimum(t_ids, n_tiles - 1)]

    idx3 = pad_pos.reshape(NW, 4, N // NW // 4)
    x_pad = _sc_scatter_rows(inp, idx3)

    nt = n_tiles.reshape(1)
    out_pad = _moe_fused(x_pad, weight1, weight2, expert_map, nt)

    return _sc_gather_rows(out_pad, pad_pos, 64)
